# speedup vs baseline: 4.7656x; 4.7656x over previous
import functools

import jax
import jax.numpy as jnp
from jax import lax
from jax.experimental import pallas as pl
from jax.experimental.pallas import tpu as pltpu
from jax.experimental.pallas import tpu_sc as plsc

BATCH = 1024
HIST = 200
FEAT = 128
VOCAB = 1000

NC, NS, L = 2, 16, 16
NW = NC * NS
TOT = BATCH * HIST
BPW = TOT // NW
CH = 128
NCH = BPW // CH

_mesh = plsc.VectorSubcoreMesh(
    core_axis_name="c", subcore_axis_name="s", num_cores=NC, num_subcores=NS
)


@functools.partial(
    pl.kernel,
    out_type=jax.ShapeDtypeStruct((TOT, FEAT), jnp.float32),
    mesh=_mesh,
    scratch_types=[
        pltpu.VMEM((BPW,), jnp.int32),
        pltpu.VMEM((CH, FEAT), jnp.float32),
        pltpu.SemaphoreType.DMA,
    ],
)
def _span_gather(idx_hbm, table_hbm, out_hbm, idx_v, rows_v, gsem):
    wid = lax.axis_index("s") * NC + lax.axis_index("c")
    base = wid * BPW

    pltpu.sync_copy(idx_hbm.at[wid], idx_v)

    @pl.loop(0, BPW // L)
    def _sub1(i):
        sl = pl.ds(i * L, L)
        idx_v[sl] = idx_v[sl] - 1

    @pl.loop(0, NCH)
    def _chunk(c):
        idx_slice = idx_v.at[pl.ds(c * CH, CH)]
        pltpu.async_copy(table_hbm.at[idx_slice], rows_v, gsem).wait()
        pltpu.sync_copy(rows_v, out_hbm.at[pl.ds(base + c * CH, CH)])


def kernel(span_width, span_width_embeddings):
    idx = span_width.reshape(NW, BPW)
    out = _span_gather(idx, span_width_embeddings)
    return out.reshape(BATCH, HIST, FEAT)


# speedup vs baseline: 5.0476x; 1.0592x over previous
import functools

import jax
import jax.numpy as jnp
from jax import lax
from jax.experimental import pallas as pl
from jax.experimental.pallas import tpu as pltpu
from jax.experimental.pallas import tpu_sc as plsc

BATCH = 1024
HIST = 200
FEAT = 128
VOCAB = 1000

NC, NS, L = 2, 16, 16
NW = NC * NS
TOT = BATCH * HIST
BPW = TOT // NW
CH = 128
NCH = BPW // CH
NBUF = 5
LAG = 2

_mesh = plsc.VectorSubcoreMesh(
    core_axis_name="c", subcore_axis_name="s", num_cores=NC, num_subcores=NS
)


@functools.partial(
    pl.kernel,
    out_type=jax.ShapeDtypeStruct((TOT, FEAT), jnp.float32),
    mesh=_mesh,
    scratch_types=[
        pltpu.VMEM((BPW,), jnp.int32),
        pltpu.VMEM((NBUF, CH, FEAT), jnp.float32),
        [pltpu.SemaphoreType.DMA] * NBUF,
        [pltpu.SemaphoreType.DMA] * NBUF,
    ],
)
def _span_gather(idx_hbm, table_hbm, out_hbm, idx_v, bufs, gsem, ssem):
    wid = lax.axis_index("s") * NC + lax.axis_index("c")
    base = wid * BPW

    def sub1_chunk(c):
        @pl.loop(0, CH // L)
        def _(j):
            sl = pl.ds(c * CH + j * L, L)
            idx_v[sl] = idx_v[sl] - 1

    def issue_gather(c, slot):
        idx_slice = idx_v.at[pl.ds(c * CH, CH)]
        pltpu.async_copy(table_hbm.at[idx_slice], bufs.at[slot], gsem[slot])

    def wait_gather(slot):
        pltpu.make_async_copy(
            table_hbm.at[pl.ds(0, CH)], bufs.at[slot], gsem[slot]
        ).wait()

    def issue_scatter(c, slot):
        pltpu.async_copy(bufs.at[slot], out_hbm.at[pl.ds(base + c * CH, CH)], ssem[slot])

    def wait_scatter(slot):
        pltpu.make_async_copy(
            bufs.at[slot], out_hbm.at[pl.ds(base, CH)], ssem[slot]
        ).wait()

    pltpu.sync_copy(idx_hbm.at[wid], idx_v)

    for b in range(NBUF):
        sub1_chunk(b)
    for b in range(NBUF):
        issue_gather(b, b)
        if b >= LAG:
            wait_gather(b - LAG)
            issue_scatter(b - LAG, b - LAG)

    @pl.loop(0, (NCH - 2 * NBUF) // NBUF)
    def _round(i):
        for b in range(NBUF):
            c = NBUF + i * NBUF + b
            sub1_chunk(c)
            wait_scatter(b)
            issue_gather(c, b)
            pb = (b - LAG) % NBUF
            wait_gather(pb)
            issue_scatter(c - LAG, pb)

    for b in range(NBUF):
        c = NCH - NBUF + b
        sub1_chunk(c)
        wait_scatter(b)
        issue_gather(c, b)
        pb = (b - LAG) % NBUF
        wait_gather(pb)
        issue_scatter(c - LAG, pb)

    for c in range(NCH - LAG, NCH):
        slot = c % NBUF
        wait_gather(slot)
        issue_scatter(c, slot)
    for b in range(NBUF):
        wait_scatter(b)


def kernel(span_width, span_width_embeddings):
    idx = span_width.reshape(NW, BPW)
    out = _span_gather(idx, span_width_embeddings)
    return out.reshape(BATCH, HIST, FEAT)


# speedup vs baseline: 11.9843x; 2.3743x over previous
import functools

import jax
import jax.numpy as jnp
from jax import lax
from jax.experimental import pallas as pl
from jax.experimental.pallas import tpu as pltpu
from jax.experimental.pallas import tpu_sc as plsc

BATCH = 1024
HIST = 200
FEAT = 128
VOCAB = 1000

NC, NS, L = 2, 16, 16
NW = NC * NS
TOT = BATCH * HIST
BPW = TOT // NW
CH = 128
NCH = BPW // CH
NBUF = 5
LAG = 2

_mesh = plsc.VectorSubcoreMesh(
    core_axis_name="c", subcore_axis_name="s", num_cores=NC, num_subcores=NS
)


@functools.partial(
    pl.kernel,
    out_type=jax.ShapeDtypeStruct((TOT, FEAT), jnp.float32),
    mesh=_mesh,
    scratch_types=[
        pltpu.VMEM((BPW,), jnp.int32),
        pltpu.VMEM((NBUF, CH, FEAT), jnp.float32),
        pltpu.VMEM_SHARED((VOCAB, FEAT), jnp.float32),
        [pltpu.SemaphoreType.DMA] * NBUF,
        [pltpu.SemaphoreType.DMA] * NBUF,
    ],
)
def _span_gather(idx_hbm, table_hbm, out_hbm, idx_v, bufs, table_sp, gsem, ssem):
    wid = lax.axis_index("s") * NC + lax.axis_index("c")
    sid = lax.axis_index("s")
    base = wid * BPW

    @pl.when(sid == 0)
    def _stage():
        pltpu.sync_copy(table_hbm, table_sp)

    plsc.subcore_barrier()

    def sub1_chunk(c):
        @pl.loop(0, CH // L)
        def _(j):
            sl = pl.ds(c * CH + j * L, L)
            idx_v[sl] = idx_v[sl] - 1

    def issue_gather(c, slot):
        idx_slice = idx_v.at[pl.ds(c * CH, CH)]
        pltpu.async_copy(table_sp.at[idx_slice], bufs.at[slot], gsem[slot])

    def wait_gather(slot):
        pltpu.make_async_copy(
            table_hbm.at[pl.ds(0, CH)], bufs.at[slot], gsem[slot]
        ).wait()

    def issue_scatter(c, slot):
        pltpu.async_copy(bufs.at[slot], out_hbm.at[pl.ds(base + c * CH, CH)], ssem[slot])

    def wait_scatter(slot):
        pltpu.make_async_copy(
            bufs.at[slot], out_hbm.at[pl.ds(base, CH)], ssem[slot]
        ).wait()

    pltpu.sync_copy(idx_hbm.at[wid], idx_v)

    for b in range(NBUF):
        sub1_chunk(b)
    for b in range(NBUF):
        issue_gather(b, b)
        if b >= LAG:
            wait_gather(b - LAG)
            issue_scatter(b - LAG, b - LAG)

    @pl.loop(0, (NCH - 2 * NBUF) // NBUF)
    def _round(i):
        for b in range(NBUF):
            c = NBUF + i * NBUF + b
            sub1_chunk(c)
            wait_scatter(b)
            issue_gather(c, b)
            pb = (b - LAG) % NBUF
            wait_gather(pb)
            issue_scatter(c - LAG, pb)

    for b in range(NBUF):
        c = NCH - NBUF + b
        sub1_chunk(c)
        wait_scatter(b)
        issue_gather(c, b)
        pb = (b - LAG) % NBUF
        wait_gather(pb)
        issue_scatter(c - LAG, pb)

    for c in range(NCH - LAG, NCH):
        slot = c % NBUF
        wait_gather(slot)
        issue_scatter(c, slot)
    for b in range(NBUF):
        wait_scatter(b)


def kernel(span_width, span_width_embeddings):
    idx = span_width.reshape(NW, BPW)
    out = _span_gather(idx, span_width_embeddings)
    return out.reshape(BATCH, HIST, FEAT)
